# baseline (speedup 1.0000x reference)
import functools

import jax
import jax.numpy as jnp
from jax import lax
from jax.experimental import pallas as pl
from jax.experimental.pallas import tpu as pltpu
from jax.experimental.pallas import tpu_sc as plsc

N = 10000
E = 320000
NC, NS = 2, 16
NW = NC * NS
CHUNK = 128
NCHUNK = 80
IDXB = 40
EPW = NCHUNK * CHUNK
EP = NW * EPW
NPAD = 10240
RPT = NPAD // NS

_mesh = functools.partial(
    plsc.VectorSubcoreMesh, core_axis_name="c", subcore_axis_name="s")


@functools.partial(
    pl.kernel,
    out_type=jax.ShapeDtypeStruct((NC, NPAD), jnp.float32),
    mesh=_mesh(),
    scratch_types=[
        pltpu.VMEM((NCHUNK, CHUNK), jnp.int32),
        pltpu.VMEM((CHUNK,), jnp.float32),
        pltpu.VMEM((RPT,), jnp.float32),
        pltpu.VMEM_SHARED((NPAD,), jnp.float32),
    ],
)
def _sc_degree(dst_hbm, out_hbm, dst_v, ones_v, buf_v, accum):
    cid = lax.axis_index("c")
    sid = lax.axis_index("s")
    wid = sid * NC + cid
    for k in range(CHUNK // 16):
        ones_v[pl.ds(k * 16, 16)] = jnp.ones((16,), jnp.float32)

    def _z(i, carry):
        buf_v[pl.ds(i * 16, 16)] = jnp.zeros((16,), jnp.float32)
        return carry

    lax.fori_loop(0, RPT // 16, _z, 0)
    pltpu.sync_copy(buf_v, accum.at[pl.ds(sid * RPT, RPT)])
    plsc.subcore_barrier()
    pltpu.sync_copy(dst_hbm.at[wid], dst_v)

    def _body(j, carry):
        pltpu.sync_copy(ones_v, accum.at[dst_v.at[j]], add=True)
        return carry

    lax.fori_loop(0, NCHUNK, _body, 0)
    plsc.subcore_barrier()
    pltpu.sync_copy(accum.at[pl.ds(sid * RPT, RPT)], buf_v)
    pltpu.sync_copy(buf_v, out_hbm.at[cid, pl.ds(sid * RPT, RPT)])


def _make_prop(w):
    @functools.partial(
        pl.kernel,
        out_type=jax.ShapeDtypeStruct((NC, NPAD, w), jnp.float32),
        mesh=_mesh(),
        compiler_params=pltpu.CompilerParams(use_tc_tiling_on_sc=(w == 128)),
        scratch_types=[
            pltpu.VMEM((IDXB, CHUNK), jnp.int32),
            pltpu.VMEM((IDXB, CHUNK), jnp.int32),
            pltpu.VMEM((CHUNK, w), jnp.float32),
            pltpu.VMEM((CHUNK, w), jnp.float32),
            pltpu.VMEM_SHARED((NPAD, w), jnp.float32),
            pltpu.SemaphoreType.DMA,
            pltpu.SemaphoreType.DMA,
        ],
    )
    def _prop(h_hbm, src_hbm, dst_hbm, out_hbm, src_v, dst_v, buf_a, buf_b,
              accum, sem_a, sem_b):
        cid = lax.axis_index("c")
        sid = lax.axis_index("s")
        wid = sid * NC + cid
        base = sid * RPT

        def _zrow(i, carry):
            for k in range(w // 16):
                buf_a[i, pl.ds(k * 16, 16)] = jnp.zeros((16,), jnp.float32)
            return carry

        lax.fori_loop(0, CHUNK, _zrow, 0)
        for t in range(RPT // CHUNK):
            pltpu.sync_copy(buf_a, accum.at[pl.ds(base + t * CHUNK, CHUNK)])
        plsc.subcore_barrier()

        def _gather(j, buf, sem):
            pltpu.async_copy(h_hbm.at[src_v.at[j]], buf, sem)

        def _gwait(j, buf, sem):
            pltpu.make_async_copy(h_hbm.at[src_v.at[j]], buf, sem).wait()

        def _scatter(j, buf):
            pltpu.sync_copy(buf, accum.at[dst_v.at[j]], add=True)

        for ph in range(NCHUNK // IDXB):
            pltpu.sync_copy(src_hbm.at[wid, pl.ds(ph * IDXB, IDXB)], src_v)
            pltpu.sync_copy(dst_hbm.at[wid, pl.ds(ph * IDXB, IDXB)], dst_v)
            _gather(0, buf_a, sem_a)

            def _body(jj, carry):
                j0 = 2 * jj
                _gather(j0 + 1, buf_b, sem_b)
                _gwait(j0, buf_a, sem_a)
                _scatter(j0, buf_a)
                _gather(j0 + 2, buf_a, sem_a)
                _gwait(j0 + 1, buf_b, sem_b)
                _scatter(j0 + 1, buf_b)
                return carry

            lax.fori_loop(0, IDXB // 2 - 1, _body, 0)
            _gather(IDXB - 1, buf_b, sem_b)
            _gwait(IDXB - 2, buf_a, sem_a)
            _scatter(IDXB - 2, buf_a)
            _gwait(IDXB - 1, buf_b, sem_b)
            _scatter(IDXB - 1, buf_b)
        plsc.subcore_barrier()
        for t in range(RPT // CHUNK):
            pltpu.sync_copy(accum.at[pl.ds(base + t * CHUNK, CHUNK)], buf_a)
            pltpu.sync_copy(buf_a, out_hbm.at[cid, pl.ds(base + t * CHUNK, CHUNK)])

    return _prop


_prop128 = _make_prop(128)
_prop64 = _make_prop(64)


R = 1000
GRID = N // R


def _row_spec(w):
    return pl.BlockSpec((R, w), lambda i: (i, 0))


def _acc_spec(w):
    return pl.BlockSpec((NC, R, w), lambda i: (0, i, 0))


def _full_spec(shape):
    return pl.BlockSpec(shape, lambda i: tuple(0 for _ in shape))


def _tc_m1(cnt_ref, x_ref, w1_ref, t1_ref, dis_ref):
    cnt = cnt_ref[...]
    dis = lax.rsqrt(cnt[:, 0:1] + cnt[:, 1:2] + 1.0)
    dis_ref[...] = dis
    t1_ref[...] = jnp.dot(x_ref[...], w1_ref[...],
                          preferred_element_type=jnp.float32) * dis


def _tc_m2(s_ref, t1_ref, dis_ref, w2_ref, wfc_ref, b1_ref, out_ref):
    dis = dis_ref[...]
    h1 = jnp.maximum(dis * (s_ref[0] + s_ref[1] + t1_ref[...]) + b1_ref[...],
                     0.0)
    w = jnp.dot(w2_ref[...], wfc_ref[...], preferred_element_type=jnp.float32)
    out_ref[...] = jnp.dot(h1, w, preferred_element_type=jnp.float32) * dis


def _tc_m3(s_ref, t2_ref, dis_ref, b2_ref, wfc_ref, bfc_ref, out_ref):
    dis = dis_ref[...]
    b = jnp.dot(b2_ref[...], wfc_ref[...],
                preferred_element_type=jnp.float32) + bfc_ref[...]
    out_ref[...] = (dis * (s_ref[0] + s_ref[1] + t2_ref[...]) + b) * dis


def _tc_m4(s_ref, z_ref, dis_ref, wd1_ref, bd1_ref, wd2_ref, out_ref):
    dis = dis_ref[...]
    u = dis * (s_ref[0] + s_ref[1] + z_ref[...])
    h3 = jnp.maximum(jnp.dot(u, wd1_ref[...],
                             preferred_element_type=jnp.float32) + bd1_ref[...],
                     0.0)
    out_ref[...] = jnp.dot(h3, wd2_ref[...],
                           preferred_element_type=jnp.float32) * dis


def _tc_m5(s_ref, t4_ref, dis_ref, wfc_ref, bd2_ref, bfc_ref, out_ref):
    dis = dis_ref[...]
    v = dis * (s_ref[0] + s_ref[1] + t4_ref[...])
    b = jnp.dot(bd2_ref[...], wfc_ref[...],
                preferred_element_type=jnp.float32) + bfc_ref[...]
    out_ref[...] = jnp.dot(v, wfc_ref[...],
                           preferred_element_type=jnp.float32) + b


def _call(body, in_specs, out_specs, out_shape):
    return pl.pallas_call(body, grid=(GRID,), in_specs=in_specs,
                          out_specs=out_specs, out_shape=out_shape)


def kernel(x, edge_index, w_e1, b_e1, w_e2, b_e2, w_efc, b_efc,
           w_d1, b_d1, w_d2, b_d2, w_dfc, b_dfc):
    pad = EP - E
    ar = jnp.arange(pad, dtype=jnp.int32)
    src_p = jnp.concatenate([edge_index[0], (ar * 37) % N]).reshape(
        NW, NCHUNK, CHUNK)
    dst_p = jnp.concatenate([edge_index[1], N + ar % (NPAD - N)]).reshape(
        NW, NCHUNK, CHUNK)

    cnt = _sc_degree(dst_p)
    cnt_t = cnt.T

    b_e1r = b_e1.reshape(1, -1)
    b_e2r = b_e2.reshape(1, -1)
    b_efcr = b_efc.reshape(1, -1)
    b_d1r = b_d1.reshape(1, -1)
    b_d2r = b_d2.reshape(1, -1)
    b_dfcr = b_dfc.reshape(1, -1)

    t1, dis = _call(
        _tc_m1,
        [pl.BlockSpec((R, 2), lambda i: (i, 0)), _row_spec(128),
         _full_spec((128, 128))],
        [_row_spec(128), _row_spec(1)],
        [jax.ShapeDtypeStruct((N, 128), jnp.float32),
         jax.ShapeDtypeStruct((N, 1), jnp.float32)],
    )(cnt_t, x, w_e1)

    s1 = _prop128(t1, src_p, dst_p)
    t2 = _call(
        _tc_m2,
        [_acc_spec(128), _row_spec(128), _row_spec(1), _full_spec((128, 64)),
         _full_spec((64, 64)), _full_spec((1, 128))],
        _row_spec(64),
        jax.ShapeDtypeStruct((N, 64), jnp.float32),
    )(s1, t1, dis, w_e2, w_efc, b_e1r)

    s2 = _prop64(t2, src_p, dst_p)
    z = _call(
        _tc_m3,
        [_acc_spec(64), _row_spec(64), _row_spec(1), _full_spec((1, 64)),
         _full_spec((64, 64)), _full_spec((1, 64))],
        _row_spec(64),
        jax.ShapeDtypeStruct((N, 64), jnp.float32),
    )(s2, t2, dis, b_e2r, w_efc, b_efcr)

    s3 = _prop64(z, src_p, dst_p)
    t4 = _call(
        _tc_m4,
        [_acc_spec(64), _row_spec(64), _row_spec(1), _full_spec((64, 256)),
         _full_spec((1, 256)), _full_spec((256, 128))],
        _row_spec(128),
        jax.ShapeDtypeStruct((N, 128), jnp.float32),
    )(s3, z, dis, w_d1, b_d1r, w_d2)

    s4 = _prop128(t4, src_p, dst_p)
    x_hat = _call(
        _tc_m5,
        [_acc_spec(128), _row_spec(128), _row_spec(1),
         _full_spec((128, 1024)), _full_spec((1, 128)), _full_spec((1, 1024))],
        _row_spec(1024),
        jax.ShapeDtypeStruct((N, 1024), jnp.float32),
    )(s4, t4, dis, w_dfc, b_d2r, b_dfcr)

    return x_hat


# speedup vs baseline: 1.0004x; 1.0004x over previous
import functools

import jax
import jax.numpy as jnp
import numpy as np
from jax import lax
from jax.experimental import pallas as pl
from jax.experimental.pallas import tpu as pltpu
from jax.experimental.pallas import tpu_sc as plsc

N = 10000
E = 320000
NC, NS = 2, 16
NW = NC * NS
CHUNK = 80
NCHUNK = 128
IDXB = 32
DEPTH = 4
EPW = NCHUNK * CHUNK
EP = NW * EPW
NPAD = 10240
RPT = NPAD // NS

_mesh = functools.partial(
    plsc.VectorSubcoreMesh, core_axis_name="c", subcore_axis_name="s")


@functools.partial(
    pl.kernel,
    out_type=jax.ShapeDtypeStruct((NC, NPAD), jnp.float32),
    mesh=_mesh(),
    scratch_types=[
        pltpu.VMEM((NCHUNK, CHUNK), jnp.int32),
        pltpu.VMEM((CHUNK,), jnp.float32),
        pltpu.VMEM((RPT,), jnp.float32),
        pltpu.VMEM_SHARED((NPAD,), jnp.float32),
    ],
)
def _sc_degree(dst_hbm, out_hbm, dst_v, ones_v, buf_v, accum):
    cid = lax.axis_index("c")
    sid = lax.axis_index("s")
    wid = sid * NC + cid
    for k in range(CHUNK // 16):
        ones_v[pl.ds(k * 16, 16)] = jnp.ones((16,), jnp.float32)

    def _z(i, carry):
        buf_v[pl.ds(i * 16, 16)] = jnp.zeros((16,), jnp.float32)
        return carry

    lax.fori_loop(0, RPT // 16, _z, 0)
    pltpu.sync_copy(buf_v, accum.at[pl.ds(sid * RPT, RPT)])
    plsc.subcore_barrier()
    pltpu.sync_copy(dst_hbm.at[wid], dst_v)

    def _body(j, carry):
        pltpu.sync_copy(ones_v, accum.at[dst_v.at[j]], add=True)
        return carry

    lax.fori_loop(0, NCHUNK, _body, 0)
    plsc.subcore_barrier()
    pltpu.sync_copy(accum.at[pl.ds(sid * RPT, RPT)], buf_v)
    pltpu.sync_copy(buf_v, out_hbm.at[cid, pl.ds(sid * RPT, RPT)])


def _make_prop(w):
    @functools.partial(
        pl.kernel,
        out_type=jax.ShapeDtypeStruct((NC, NPAD, w), jnp.float32),
        mesh=_mesh(),
        compiler_params=pltpu.CompilerParams(use_tc_tiling_on_sc=(w == 128)),
        scratch_types=[
            pltpu.VMEM((IDXB, CHUNK), jnp.int32),
            pltpu.VMEM((IDXB, CHUNK), jnp.int32),
        ] + [pltpu.VMEM((CHUNK, w), jnp.float32) for _ in range(DEPTH)]
        + [pltpu.VMEM_SHARED((NPAD, w), jnp.float32)]
        + [pltpu.SemaphoreType.DMA for _ in range(DEPTH)],
    )
    def _prop(h_hbm, src_hbm, dst_hbm, out_hbm, src_v, dst_v,
              b0, b1, b2, b3, accum, s0, s1, s2, s3):
        bufs = (b0, b1, b2, b3)
        sems = (s0, s1, s2, s3)
        cid = lax.axis_index("c")
        sid = lax.axis_index("s")
        wid = sid * NC + cid
        base = sid * RPT

        def _zrow(i, carry):
            for k in range(w // 16):
                b0[i, pl.ds(k * 16, 16)] = jnp.zeros((16,), jnp.float32)
            return carry

        lax.fori_loop(0, CHUNK, _zrow, 0)
        for t in range(RPT // CHUNK):
            pltpu.sync_copy(b0, accum.at[pl.ds(base + t * CHUNK, CHUNK)])
        plsc.subcore_barrier()

        def _gather(j, t):
            pltpu.async_copy(h_hbm.at[src_v.at[j]], bufs[t], sems[t])

        def _gwait(j, t):
            pltpu.make_async_copy(h_hbm.at[src_v.at[j]], bufs[t],
                                  sems[t]).wait()

        def _scatter(j, t):
            pltpu.sync_copy(bufs[t], accum.at[dst_v.at[j]], add=True)

        for ph in range(NCHUNK // IDXB):
            pltpu.sync_copy(src_hbm.at[wid, pl.ds(ph * IDXB, IDXB)], src_v)
            pltpu.sync_copy(dst_hbm.at[wid, pl.ds(ph * IDXB, IDXB)], dst_v)
            for t in range(DEPTH - 1):
                _gather(t, t)

            def _body(kk, carry):
                j0 = DEPTH * kk
                for t in range(DEPTH):
                    _gwait(j0 + t, t)
                    _scatter(j0 + t, t)
                    _gather(j0 + t + DEPTH - 1, (t + DEPTH - 1) % DEPTH)
                return carry

            lax.fori_loop(0, IDXB // DEPTH - 1, _body, 0)
            j0 = IDXB - DEPTH
            _gather(IDXB - 1, (IDXB - 1) % DEPTH)
            for t in range(DEPTH):
                _gwait(j0 + t, t)
                _scatter(j0 + t, t)
        plsc.subcore_barrier()
        for t in range(RPT // CHUNK):
            pltpu.sync_copy(accum.at[pl.ds(base + t * CHUNK, CHUNK)], b0)
            pltpu.sync_copy(b0, out_hbm.at[cid, pl.ds(base + t * CHUNK, CHUNK)])

    return _prop


_prop128 = _make_prop(128)
_prop64 = _make_prop(64)


R = 1000
GRID = N // R


def _row_spec(w):
    return pl.BlockSpec((R, w), lambda i: (i, 0))


def _acc_spec(w):
    return pl.BlockSpec((NC, R, w), lambda i: (0, i, 0))


def _full_spec(shape):
    return pl.BlockSpec(shape, lambda i: tuple(0 for _ in shape))


def _tc_m1(cnt_ref, x_ref, w1_ref, t1_ref, dis_ref):
    cnt = cnt_ref[...]
    dis = lax.rsqrt(cnt[:, 0:1] + cnt[:, 1:2] + 1.0)
    dis_ref[...] = dis
    t1_ref[...] = jnp.dot(x_ref[...], w1_ref[...],
                          preferred_element_type=jnp.float32) * dis


def _tc_m2(s_ref, t1_ref, dis_ref, w2_ref, wfc_ref, b1_ref, out_ref):
    dis = dis_ref[...]
    h1 = jnp.maximum(dis * (s_ref[0] + s_ref[1] + t1_ref[...]) + b1_ref[...],
                     0.0)
    w = jnp.dot(w2_ref[...], wfc_ref[...], preferred_element_type=jnp.float32)
    out_ref[...] = jnp.dot(h1, w, preferred_element_type=jnp.float32) * dis


def _tc_m3(s_ref, t2_ref, dis_ref, b2_ref, wfc_ref, bfc_ref, out_ref):
    dis = dis_ref[...]
    b = jnp.dot(b2_ref[...], wfc_ref[...],
                preferred_element_type=jnp.float32) + bfc_ref[...]
    out_ref[...] = (dis * (s_ref[0] + s_ref[1] + t2_ref[...]) + b) * dis


def _tc_m4(s_ref, z_ref, dis_ref, wd1_ref, bd1_ref, wd2_ref, out_ref):
    dis = dis_ref[...]
    u = dis * (s_ref[0] + s_ref[1] + z_ref[...])
    h3 = jnp.maximum(jnp.dot(u, wd1_ref[...],
                             preferred_element_type=jnp.float32) + bd1_ref[...],
                     0.0)
    out_ref[...] = jnp.dot(h3, wd2_ref[...],
                           preferred_element_type=jnp.float32) * dis


def _tc_m5(s_ref, t4_ref, dis_ref, wfc_ref, bd2_ref, bfc_ref, out_ref):
    dis = dis_ref[...]
    v = dis * (s_ref[0] + s_ref[1] + t4_ref[...])
    b = jnp.dot(bd2_ref[...], wfc_ref[...],
                preferred_element_type=jnp.float32) + bfc_ref[...]
    out_ref[...] = jnp.dot(v, wfc_ref[...],
                           preferred_element_type=jnp.float32) + b


def _call(body, in_specs, out_specs, out_shape):
    return pl.pallas_call(body, grid=(GRID,), in_specs=in_specs,
                          out_specs=out_specs, out_shape=out_shape)


def kernel(x, edge_index, w_e1, b_e1, w_e2, b_e2, w_efc, b_efc,
           w_d1, b_d1, w_d2, b_d2, w_dfc, b_dfc):
    ar = np.arange(EP - E, dtype=np.int32)
    pad_src = jnp.asarray((ar * 37) % N, dtype=jnp.int32)
    pad_dst = jnp.asarray(N + ar % (NPAD - N), dtype=jnp.int32)
    src_p = jnp.concatenate([edge_index[0], pad_src]).reshape(
        NW, NCHUNK, CHUNK)
    dst_p = jnp.concatenate([edge_index[1], pad_dst]).reshape(
        NW, NCHUNK, CHUNK)

    cnt = _sc_degree(dst_p)
    cnt_t = cnt.T

    b_e1r = b_e1.reshape(1, -1)
    b_e2r = b_e2.reshape(1, -1)
    b_efcr = b_efc.reshape(1, -1)
    b_d1r = b_d1.reshape(1, -1)
    b_d2r = b_d2.reshape(1, -1)
    b_dfcr = b_dfc.reshape(1, -1)

    t1, dis = _call(
        _tc_m1,
        [pl.BlockSpec((R, 2), lambda i: (i, 0)), _row_spec(128),
         _full_spec((128, 128))],
        [_row_spec(128), _row_spec(1)],
        [jax.ShapeDtypeStruct((N, 128), jnp.float32),
         jax.ShapeDtypeStruct((N, 1), jnp.float32)],
    )(cnt_t, x, w_e1)

    s1 = _prop128(t1, src_p, dst_p)
    t2 = _call(
        _tc_m2,
        [_acc_spec(128), _row_spec(128), _row_spec(1), _full_spec((128, 64)),
         _full_spec((64, 64)), _full_spec((1, 128))],
        _row_spec(64),
        jax.ShapeDtypeStruct((N, 64), jnp.float32),
    )(s1, t1, dis, w_e2, w_efc, b_e1r)

    s2 = _prop64(t2, src_p, dst_p)
    z = _call(
        _tc_m3,
        [_acc_spec(64), _row_spec(64), _row_spec(1), _full_spec((1, 64)),
         _full_spec((64, 64)), _full_spec((1, 64))],
        _row_spec(64),
        jax.ShapeDtypeStruct((N, 64), jnp.float32),
    )(s2, t2, dis, b_e2r, w_efc, b_efcr)

    s3 = _prop64(z, src_p, dst_p)
    t4 = _call(
        _tc_m4,
        [_acc_spec(64), _row_spec(64), _row_spec(1), _full_spec((64, 256)),
         _full_spec((1, 256)), _full_spec((256, 128))],
        _row_spec(128),
        jax.ShapeDtypeStruct((N, 128), jnp.float32),
    )(s3, z, dis, w_d1, b_d1r, w_d2)

    s4 = _prop128(t4, src_p, dst_p)
    x_hat = _call(
        _tc_m5,
        [_acc_spec(128), _row_spec(128), _row_spec(1),
         _full_spec((128, 1024)), _full_spec((1, 128)), _full_spec((1, 1024))],
        _row_spec(1024),
        jax.ShapeDtypeStruct((N, 1024), jnp.float32),
    )(s4, t4, dis, w_dfc, b_d2r, b_dfcr)

    return x_hat


# speedup vs baseline: 1.0215x; 1.0210x over previous
import functools

import jax
import jax.numpy as jnp
import numpy as np
from jax import lax
from jax.experimental import pallas as pl
from jax.experimental.pallas import tpu as pltpu
from jax.experimental.pallas import tpu_sc as plsc

N = 10000
E = 320000
NC, NS = 2, 16
NW = NC * NS
CHUNK = 80
NCHUNK = 128
IDXB = 32
DEPTH = 4
EPW = NCHUNK * CHUNK
EP = NW * EPW
NPAD = 10240
RPT = NPAD // NS

_mesh = functools.partial(
    plsc.VectorSubcoreMesh, core_axis_name="c", subcore_axis_name="s")


@functools.partial(
    pl.kernel,
    out_type=jax.ShapeDtypeStruct((NC, NPAD), jnp.float32),
    mesh=_mesh(),
    scratch_types=[
        pltpu.VMEM((NCHUNK, CHUNK), jnp.int32),
        pltpu.VMEM((CHUNK,), jnp.float32),
        pltpu.VMEM((RPT,), jnp.float32),
        pltpu.VMEM_SHARED((NPAD,), jnp.float32),
    ],
)
def _sc_degree(dst_hbm, out_hbm, dst_v, ones_v, buf_v, accum):
    cid = lax.axis_index("c")
    sid = lax.axis_index("s")
    wid = sid * NC + cid
    for k in range(CHUNK // 16):
        ones_v[pl.ds(k * 16, 16)] = jnp.ones((16,), jnp.float32)

    def _z(i, carry):
        buf_v[pl.ds(i * 16, 16)] = jnp.zeros((16,), jnp.float32)
        return carry

    lax.fori_loop(0, RPT // 16, _z, 0)
    pltpu.sync_copy(buf_v, accum.at[pl.ds(sid * RPT, RPT)])
    plsc.subcore_barrier()
    pltpu.sync_copy(dst_hbm.at[wid], dst_v)

    def _body(j, carry):
        pltpu.sync_copy(ones_v, accum.at[dst_v.at[j]], add=True)
        return carry

    lax.fori_loop(0, NCHUNK, _body, 0)
    plsc.subcore_barrier()
    pltpu.sync_copy(accum.at[pl.ds(sid * RPT, RPT)], buf_v)
    pltpu.sync_copy(buf_v, out_hbm.at[cid, pl.ds(sid * RPT, RPT)])


def _make_prop(w):
    @functools.partial(
        pl.kernel,
        out_type=jax.ShapeDtypeStruct((NC, NPAD, w), jnp.float32),
        mesh=_mesh(),
        compiler_params=pltpu.CompilerParams(use_tc_tiling_on_sc=(w == 128)),
        scratch_types=[
            pltpu.VMEM((IDXB, CHUNK), jnp.int32),
            pltpu.VMEM((IDXB, CHUNK), jnp.int32),
        ] + [pltpu.VMEM((CHUNK, w), jnp.float32) for _ in range(DEPTH)]
        + [pltpu.VMEM_SHARED((NPAD, w), jnp.float32)]
        + [pltpu.SemaphoreType.DMA for _ in range(DEPTH)],
    )
    def _prop(h_hbm, src_hbm, dst_hbm, out_hbm, src_v, dst_v,
              b0, b1, b2, b3, accum, s0, s1, s2, s3):
        bufs = (b0, b1, b2, b3)
        sems = (s0, s1, s2, s3)
        cid = lax.axis_index("c")
        sid = lax.axis_index("s")
        wid = sid * NC + cid
        base = sid * RPT

        def _zrow(i, carry):
            for k in range(w // 16):
                b0[i, pl.ds(k * 16, 16)] = jnp.zeros((16,), jnp.float32)
            return carry

        lax.fori_loop(0, CHUNK, _zrow, 0)
        for t in range(RPT // CHUNK):
            pltpu.sync_copy(b0, accum.at[pl.ds(base + t * CHUNK, CHUNK)])
        plsc.subcore_barrier()

        def _gather(j, t):
            pltpu.async_copy(h_hbm.at[src_v.at[j]], bufs[t], sems[t])

        def _gwait(j, t):
            pltpu.make_async_copy(h_hbm.at[src_v.at[j]], bufs[t],
                                  sems[t]).wait()

        def _scatter(j, t):
            pltpu.sync_copy(bufs[t], accum.at[dst_v.at[j]], add=True)

        for ph in range(NCHUNK // IDXB):
            pltpu.sync_copy(src_hbm.at[wid, pl.ds(ph * IDXB, IDXB)], src_v)
            pltpu.sync_copy(dst_hbm.at[wid, pl.ds(ph * IDXB, IDXB)], dst_v)
            for t in range(DEPTH - 1):
                _gather(t, t)

            def _body(kk, carry):
                j0 = DEPTH * kk
                for t in range(DEPTH):
                    _gwait(j0 + t, t)
                    _scatter(j0 + t, t)
                    _gather(j0 + t + DEPTH - 1, (t + DEPTH - 1) % DEPTH)
                return carry

            lax.fori_loop(0, IDXB // DEPTH - 1, _body, 0)
            j0 = IDXB - DEPTH
            _gather(IDXB - 1, (IDXB - 1) % DEPTH)
            for t in range(DEPTH):
                _gwait(j0 + t, t)
                _scatter(j0 + t, t)
        plsc.subcore_barrier()
        for t in range(RPT // CHUNK):
            pltpu.sync_copy(accum.at[pl.ds(base + t * CHUNK, CHUNK)], b0)
            pltpu.sync_copy(b0, out_hbm.at[cid, pl.ds(base + t * CHUNK, CHUNK)])

    return _prop


_prop128 = _make_prop(128)
_prop64 = _make_prop(64)


R = 2000
GRID = N // R


def _row_spec(w):
    return pl.BlockSpec((R, w), lambda i: (i, 0))


def _acc_spec(w):
    return pl.BlockSpec((NC, R, w), lambda i: (0, i, 0))


def _full_spec(shape):
    return pl.BlockSpec(shape, lambda i: tuple(0 for _ in shape))


def _tc_m1(cnt_ref, x_ref, w1_ref, t1_ref, dis_ref):
    cnt = cnt_ref[...]
    dis = lax.rsqrt(cnt[:, 0:1] + cnt[:, 1:2] + 1.0)
    dis_ref[...] = dis
    t1_ref[...] = jnp.dot(x_ref[...], w1_ref[...],
                          preferred_element_type=jnp.float32) * dis


def _tc_m2(s_ref, t1_ref, dis_ref, w2_ref, wfc_ref, b1_ref, out_ref):
    dis = dis_ref[...]
    h1 = jnp.maximum(dis * (s_ref[0] + s_ref[1] + t1_ref[...]) + b1_ref[...],
                     0.0)
    w = jnp.dot(w2_ref[...], wfc_ref[...], preferred_element_type=jnp.float32)
    out_ref[...] = jnp.dot(h1, w, preferred_element_type=jnp.float32) * dis


def _tc_m3(s_ref, t2_ref, dis_ref, b2_ref, wfc_ref, bfc_ref, out_ref):
    dis = dis_ref[...]
    b = jnp.dot(b2_ref[...], wfc_ref[...],
                preferred_element_type=jnp.float32) + bfc_ref[...]
    out_ref[...] = (dis * (s_ref[0] + s_ref[1] + t2_ref[...]) + b) * dis


def _tc_m4(s_ref, z_ref, dis_ref, wd1_ref, bd1_ref, wd2_ref, out_ref):
    dis = dis_ref[...]
    u = dis * (s_ref[0] + s_ref[1] + z_ref[...])
    h3 = jnp.maximum(jnp.dot(u, wd1_ref[...],
                             preferred_element_type=jnp.float32) + bd1_ref[...],
                     0.0)
    out_ref[...] = jnp.dot(h3, wd2_ref[...],
                           preferred_element_type=jnp.float32) * dis


def _tc_m5(s_ref, t4_ref, dis_ref, wfc_ref, bd2_ref, bfc_ref, out_ref):
    dis = dis_ref[...]
    v = dis * (s_ref[0] + s_ref[1] + t4_ref[...])
    b = jnp.dot(bd2_ref[...], wfc_ref[...],
                preferred_element_type=jnp.float32) + bfc_ref[...]
    out_ref[...] = jnp.dot(v, wfc_ref[...],
                           preferred_element_type=jnp.float32) + b


def _call(body, in_specs, out_specs, out_shape):
    return pl.pallas_call(body, grid=(GRID,), in_specs=in_specs,
                          out_specs=out_specs, out_shape=out_shape)


def kernel(x, edge_index, w_e1, b_e1, w_e2, b_e2, w_efc, b_efc,
           w_d1, b_d1, w_d2, b_d2, w_dfc, b_dfc):
    ar = np.arange(EP - E, dtype=np.int32)
    pad_src = jnp.asarray((ar * 37) % N, dtype=jnp.int32)
    pad_dst = jnp.asarray(N + ar % (NPAD - N), dtype=jnp.int32)
    src_p = jnp.concatenate([edge_index[0], pad_src]).reshape(
        NW, NCHUNK, CHUNK)
    dst_p = jnp.concatenate([edge_index[1], pad_dst]).reshape(
        NW, NCHUNK, CHUNK)

    cnt = _sc_degree(dst_p)
    cnt_t = cnt.T

    b_e1r = b_e1.reshape(1, -1)
    b_e2r = b_e2.reshape(1, -1)
    b_efcr = b_efc.reshape(1, -1)
    b_d1r = b_d1.reshape(1, -1)
    b_d2r = b_d2.reshape(1, -1)
    b_dfcr = b_dfc.reshape(1, -1)

    t1, dis = _call(
        _tc_m1,
        [pl.BlockSpec((R, 2), lambda i: (i, 0)), _row_spec(128),
         _full_spec((128, 128))],
        [_row_spec(128), _row_spec(1)],
        [jax.ShapeDtypeStruct((N, 128), jnp.float32),
         jax.ShapeDtypeStruct((N, 1), jnp.float32)],
    )(cnt_t, x, w_e1)

    s1 = _prop128(t1, src_p, dst_p)
    t2 = _call(
        _tc_m2,
        [_acc_spec(128), _row_spec(128), _row_spec(1), _full_spec((128, 64)),
         _full_spec((64, 64)), _full_spec((1, 128))],
        _row_spec(64),
        jax.ShapeDtypeStruct((N, 64), jnp.float32),
    )(s1, t1, dis, w_e2, w_efc, b_e1r)

    s2 = _prop64(t2, src_p, dst_p)
    z = _call(
        _tc_m3,
        [_acc_spec(64), _row_spec(64), _row_spec(1), _full_spec((1, 64)),
         _full_spec((64, 64)), _full_spec((1, 64))],
        _row_spec(64),
        jax.ShapeDtypeStruct((N, 64), jnp.float32),
    )(s2, t2, dis, b_e2r, w_efc, b_efcr)

    s3 = _prop64(z, src_p, dst_p)
    t4 = _call(
        _tc_m4,
        [_acc_spec(64), _row_spec(64), _row_spec(1), _full_spec((64, 256)),
         _full_spec((1, 256)), _full_spec((256, 128))],
        _row_spec(128),
        jax.ShapeDtypeStruct((N, 128), jnp.float32),
    )(s3, z, dis, w_d1, b_d1r, w_d2)

    s4 = _prop128(t4, src_p, dst_p)
    x_hat = _call(
        _tc_m5,
        [_acc_spec(128), _row_spec(128), _row_spec(1),
         _full_spec((128, 1024)), _full_spec((1, 128)), _full_spec((1, 1024))],
        _row_spec(1024),
        jax.ShapeDtypeStruct((N, 1024), jnp.float32),
    )(s4, t4, dis, w_dfc, b_d2r, b_dfcr)

    return x_hat
